# Initial kernel scaffold; baseline (speedup 1.0000x reference)
import jax
import jax.numpy as jnp
from jax.experimental import pallas as pl


def kernel(input_ids, token_type_ids, tok_table, pos_table, seg_table, gamma, beta):
    raise NotImplementedError("write your pallas kernel here")


# speedup vs baseline: 3.1445x; 3.1445x over previous
import functools

import jax
import jax.numpy as jnp
from jax import lax
from jax.experimental import pallas as pl
from jax.experimental.pallas import tpu as pltpu
from jax.experimental.pallas import tpu_sc as plsc

NC = 2
NS = 16
NW = NC * NS
L = 16
C = 128


_GATHER_DNUMS = lax.GatherDimensionNumbers(
    offset_dims=(), collapsed_slice_dims=(0,), start_index_map=(0,))


def _permute16(v, p):
    return lax.gather(v, p[:, None], _GATHER_DNUMS, slice_sizes=(1,),
                      mode=lax.GatherScatterMode.PROMISE_IN_BOUNDS)


def _hsum16(v, perms):
    for p in perms:
        v = v + _permute16(v, p)
    return v


def _rsqrt16(x):
    i = lax.bitcast_convert_type(x, jnp.int32)
    y = lax.bitcast_convert_type(jnp.int32(0x5F3759DF) - (i >> 1), jnp.float32)
    for _ in range(3):
        y = y * (1.5 - 0.5 * x * y * y)
    return y


@functools.partial(jax.jit, static_argnums=(6, 7))
def _run(ids_flat, cidx_flat, tok_table, ps_table, gamma, beta, n_tokens, hidden):
    H = hidden
    J = H // L
    tpw = n_tokens // NW
    nchunk = tpw // C

    mesh = plsc.VectorSubcoreMesh(core_axis_name="c", subcore_axis_name="s")

    @functools.partial(
        pl.kernel,
        mesh=mesh,
        out_type=jax.ShapeDtypeStruct((n_tokens, H), jnp.float32),
        scratch_types=[
            pltpu.VMEM((C,), jnp.int32),
            pltpu.VMEM((C,), jnp.int32),
            pltpu.VMEM((C, H), jnp.float32),
            pltpu.VMEM((C, H), jnp.float32),
            pltpu.VMEM((H,), jnp.float32),
            pltpu.VMEM((H,), jnp.float32),
            pltpu.SemaphoreType.DMA,
            pltpu.SemaphoreType.DMA,
        ],
    )
    def sc_kernel(ids_hbm, cidx_hbm, tok_hbm, ps_hbm, gamma_hbm, beta_hbm,
                  out_hbm, idx_v, cidx_v, rowT, rowP, gamma_v, beta_v,
                  semT, semP):
        wid = lax.axis_index("s") * NC + lax.axis_index("c")
        base = wid * tpw
        pltpu.sync_copy(gamma_hbm, gamma_v)
        pltpu.sync_copy(beta_hbm, beta_v)
        g8 = [gamma_v[pl.ds(L * j, L)] for j in range(J)]
        b8 = [beta_v[pl.ds(L * j, L)] for j in range(J)]
        lane = lax.iota(jnp.int32, L)
        perms = [lane ^ s for s in (8, 4, 2, 1)]

        def chunk_body(g, carry):
            off = base + g * C
            pltpu.sync_copy(ids_hbm.at[pl.ds(off, C)], idx_v)
            pltpu.sync_copy(cidx_hbm.at[pl.ds(off, C)], cidx_v)
            cpT = pltpu.async_copy(tok_hbm.at[idx_v], rowT, semT)
            cpP = pltpu.async_copy(ps_hbm.at[cidx_v], rowP, semP)
            cpT.wait()
            cpP.wait()

            def tok_body(t, tc):
                vs = []
                acc = None
                accsq = None
                for j in range(J):
                    v = rowT[t, pl.ds(L * j, L)] + rowP[t, pl.ds(L * j, L)]
                    vs.append(v)
                    acc = v if acc is None else acc + v
                    accsq = v * v if accsq is None else accsq + v * v
                mean_v = _hsum16(acc, perms) * (1.0 / H)
                var_v = _hsum16(accsq, perms) * (1.0 / H) - mean_v * mean_v
                inv_v = _rsqrt16(var_v + 1e-5)
                for j in range(J):
                    rowT[t, pl.ds(L * j, L)] = ((vs[j] - mean_v) * inv_v
                                                * g8[j] + b8[j])
                return tc

            lax.fori_loop(0, C, tok_body, 0)
            pltpu.sync_copy(rowT, out_hbm.at[pl.ds(off, C)])
            return carry

        lax.fori_loop(0, nchunk, chunk_body, 0)

    return sc_kernel(ids_flat, cidx_flat, tok_table, ps_table, gamma, beta)


def kernel(input_ids, token_type_ids, tok_table, pos_table, seg_table, gamma, beta):
    B, S = input_ids.shape
    H = tok_table.shape[1]
    TV = seg_table.shape[0]
    n = B * S
    ps_table = (pos_table[:S, None, :] + seg_table[None, :, :]).reshape(S * TV, H)
    pos_ids = jnp.arange(S, dtype=jnp.int32)
    cidx = (pos_ids[None, :] * TV + token_type_ids.astype(jnp.int32)).reshape(-1)
    ids_flat = input_ids.astype(jnp.int32).reshape(-1)
    out = _run(ids_flat, cidx, tok_table, ps_table, gamma, beta, n, H)
    return out.reshape(B, S, H)


# speedup vs baseline: 3.5340x; 1.1239x over previous
import functools

import jax
import jax.numpy as jnp
from jax import lax
from jax.experimental import pallas as pl
from jax.experimental.pallas import tpu as pltpu
from jax.experimental.pallas import tpu_sc as plsc

NC = 2
NS = 16
NW = NC * NS
L = 16
CH = 128
NB = 4

_GATHER_DNUMS = lax.GatherDimensionNumbers(
    offset_dims=(), collapsed_slice_dims=(0,), start_index_map=(0,))


def _permute16(v, p):
    return lax.gather(v, p[:, None], _GATHER_DNUMS, slice_sizes=(1,),
                      mode=lax.GatherScatterMode.PROMISE_IN_BOUNDS)


def _hsum16(v, perms):
    for p in perms:
        v = v + _permute16(v, p)
    return v


def _rsqrt16(x):
    i = lax.bitcast_convert_type(x, jnp.int32)
    y = lax.bitcast_convert_type(jnp.int32(0x5F3759DF) - (i >> 1), jnp.float32)
    for _ in range(3):
        y = y * (1.5 - 0.5 * x * y * y)
    return y


@functools.partial(jax.jit, static_argnums=(7, 8, 9))
def _run(ids2, ttf3, tok_table, posadj, dseg, gamma, beta,
         n_tokens, hidden, seq):
    H = hidden
    J = H // L
    tpw = n_tokens // NW
    nchunk = tpw // CH
    nrows = n_tokens // CH

    mesh = plsc.VectorSubcoreMesh(core_axis_name="c", subcore_axis_name="s")

    @functools.partial(
        pl.kernel,
        mesh=mesh,
        out_type=jax.ShapeDtypeStruct((nrows, CH, H), jnp.float32),
        scratch_types=[
            pltpu.VMEM((NB, CH), jnp.int32),
            pltpu.VMEM((NB, CH // L, L), jnp.float32),
            pltpu.VMEM((NB, CH, H), jnp.float32),
            pltpu.VMEM((seq, H), jnp.float32),
            pltpu.VMEM((H,), jnp.float32),
            pltpu.VMEM((H,), jnp.float32),
            pltpu.VMEM((H,), jnp.float32),
        ] + [pltpu.SemaphoreType.DMA] * (2 * NB),
    )
    def sc_kernel(ids_hbm, ttf_hbm, tok_hbm, posadj_hbm, dseg_hbm,
                  gamma_hbm, beta_hbm, out_hbm,
                  idx_all, ttf_all, rowbuf, posadj_v, dseg_v, gamma_v, beta_v,
                  *sems):
        gsem = sems[:NB]
        osem = sems[NB:]
        wid = lax.axis_index("s") * NC + lax.axis_index("c")
        row0 = wid * nchunk
        pltpu.sync_copy(posadj_hbm, posadj_v)
        pltpu.sync_copy(dseg_hbm, dseg_v)
        pltpu.sync_copy(gamma_hbm, gamma_v)
        pltpu.sync_copy(beta_hbm, beta_v)
        g8 = [gamma_v[pl.ds(L * j, L)] for j in range(J)]
        b8 = [beta_v[pl.ds(L * j, L)] for j in range(J)]
        d8 = [dseg_v[pl.ds(L * j, L)] for j in range(J)]
        lane = lax.iota(jnp.int32, L)
        perms = [lane ^ s for s in (8, 4, 2, 1)]

        def prep(g3, b3):
            r = row0 + g3
            pltpu.sync_copy(ids_hbm.at[r], idx_all.at[b3])
            pltpu.sync_copy(ttf_hbm.at[r], ttf_all.at[b3])
            pltpu.async_copy(tok_hbm.at[idx_all.at[b3]], rowbuf.at[b3],
                             gsem[b3])

        def gwait(b):
            pltpu.make_async_copy(tok_hbm.at[idx_all.at[b]], rowbuf.at[b],
                                  gsem[b]).wait()

        def owait(b):
            pltpu.make_async_copy(rowbuf.at[b], out_hbm.at[0],
                                  osem[b]).wait()

        def compute(b, g):
            p0 = lax.rem(g * CH, seq)

            def tok_body(t, carry):
                pt = p0 + t
                p = pt - jnp.where(pt >= seq, seq, 0)
                tg = ttf_all[b, t >> 4, :]
                tts = _permute16(tg, jnp.broadcast_to(t & (L - 1), (L,)))
                vs = []
                acc = None
                accsq = None
                for j in range(J):
                    v = (rowbuf[b, t, pl.ds(L * j, L)]
                         + posadj_v[p, pl.ds(L * j, L)]
                         + tts * d8[j])
                    vs.append(v)
                    acc = v if acc is None else acc + v
                    accsq = v * v if accsq is None else accsq + v * v
                mean_v = _hsum16(acc, perms) * (1.0 / H)
                var_v = _hsum16(accsq, perms) * (1.0 / H) - mean_v * mean_v
                inv_v = _rsqrt16(var_v + 1e-5)
                for j in range(J):
                    rowbuf[b, t, pl.ds(L * j, L)] = ((vs[j] - mean_v) * inv_v
                                                     * g8[j] + b8[j])
                return carry

            lax.fori_loop(0, CH, tok_body, 0)
            pltpu.async_copy(rowbuf.at[b], out_hbm.at[row0 + g], osem[b])

        def step(g, b, do_issue, do_owait):
            if do_issue:
                b3 = (b + 3) % NB
                if do_owait:
                    owait(b3)
                prep(g + 3, b3)
            gwait(b)
            compute(b, g)

        for gg in range(NB - 1):
            prep(gg, gg)
        step(0, 0, True, False)
        step(1, 1, True, True)
        step(2, 2, True, True)
        step(3, 3, True, True)

        def outer(go, carry):
            g0 = go * NB
            for b in range(NB):
                step(g0 + b, b, True, True)
            return carry

        ngroups = nchunk // NB
        lax.fori_loop(1, ngroups - 1, outer, 0)

        for g in range((ngroups - 1) * NB, nchunk):
            b = g % NB
            step(g, b, g + 3 < nchunk, True)
        for b in range(NB):
            owait(b)

    return sc_kernel(ids2, ttf3, tok_table, posadj, dseg, gamma, beta)


def kernel(input_ids, token_type_ids, tok_table, pos_table, seg_table, gamma, beta):
    B, S = input_ids.shape
    H = tok_table.shape[1]
    n = B * S
    nrows = n // CH
    ids2 = input_ids.astype(jnp.int32).reshape(nrows, CH)
    ttf3 = token_type_ids.astype(jnp.float32).reshape(nrows, CH // L, L)
    posadj = pos_table[:S] + seg_table[0][None, :]
    dseg = seg_table[1] - seg_table[0]
    out = _run(ids2, ttf3, tok_table, posadj, dseg, gamma, beta, n, H, S)
    return out.reshape(B, S, H)


# speedup vs baseline: 4.6822x; 1.3249x over previous
import functools

import jax
import jax.numpy as jnp
from jax import lax
from jax.experimental import pallas as pl
from jax.experimental.pallas import tpu as pltpu
from jax.experimental.pallas import tpu_sc as plsc

NC = 2
NS = 16
NW = NC * NS
L = 16
CH = 128
NB = 4

_GATHER_DNUMS = lax.GatherDimensionNumbers(
    offset_dims=(), collapsed_slice_dims=(0,), start_index_map=(0,))


def _permute16(v, p):
    return lax.gather(v, p[:, None], _GATHER_DNUMS, slice_sizes=(1,),
                      mode=lax.GatherScatterMode.PROMISE_IN_BOUNDS)


def _hsum16(v, perms):
    for p in perms:
        v = v + _permute16(v, p)
    return v


def _rsqrt16(x):
    i = lax.bitcast_convert_type(x, jnp.int32)
    y = lax.bitcast_convert_type(jnp.int32(0x5F3759DF) - (i >> 1), jnp.float32)
    for _ in range(3):
        y = y * (1.5 - 0.5 * x * y * y)
    return y


@functools.partial(jax.jit, static_argnums=(7, 8, 9))
def _run(ids2, ttf3, tok_table, posadj, dseg, gamma, beta,
         n_tokens, hidden, seq):
    H = hidden
    J = H // L
    tpw = n_tokens // NW
    nchunk = tpw // CH
    nrows = n_tokens // CH

    mesh = plsc.VectorSubcoreMesh(core_axis_name="c", subcore_axis_name="s")

    @functools.partial(
        pl.kernel,
        mesh=mesh,
        out_type=jax.ShapeDtypeStruct((nrows, CH, H), jnp.float32),
        scratch_types=[
            pltpu.VMEM((NB, CH), jnp.int32),
            pltpu.VMEM((NB, CH // L, L), jnp.float32),
            pltpu.VMEM((NB, CH, H), jnp.float32),
            pltpu.VMEM((seq, H), jnp.float32),
            pltpu.VMEM((H,), jnp.float32),
            pltpu.VMEM((H,), jnp.float32),
            pltpu.VMEM((H,), jnp.float32),
        ] + [pltpu.SemaphoreType.DMA] * (2 * NB),
    )
    def sc_kernel(ids_hbm, ttf_hbm, tok_hbm, posadj_hbm, dseg_hbm,
                  gamma_hbm, beta_hbm, out_hbm,
                  idx_all, ttf_all, rowbuf, posadj_v, dseg_v, gamma_v, beta_v,
                  *sems):
        gsem = sems[:NB]
        osem = sems[NB:]
        wid = lax.axis_index("s") * NC + lax.axis_index("c")
        row0 = wid * nchunk
        pltpu.sync_copy(posadj_hbm, posadj_v)
        pltpu.sync_copy(dseg_hbm, dseg_v)
        pltpu.sync_copy(gamma_hbm, gamma_v)
        pltpu.sync_copy(beta_hbm, beta_v)
        g8 = [gamma_v[pl.ds(L * j, L)] for j in range(J)]
        b8 = [beta_v[pl.ds(L * j, L)] for j in range(J)]
        d8 = [dseg_v[pl.ds(L * j, L)] for j in range(J)]
        lane = lax.iota(jnp.int32, L)
        perms = [lane ^ s for s in (8, 4, 2, 1)]

        def prep(g3, b3):
            r = row0 + g3
            pltpu.sync_copy(ids_hbm.at[r], idx_all.at[b3])
            pltpu.sync_copy(ttf_hbm.at[r], ttf_all.at[b3])
            pltpu.async_copy(tok_hbm.at[idx_all.at[b3]], rowbuf.at[b3],
                             gsem[b3])

        def gwait(b):
            pltpu.make_async_copy(tok_hbm.at[idx_all.at[b]], rowbuf.at[b],
                                  gsem[b]).wait()

        def owait(b):
            pltpu.make_async_copy(rowbuf.at[b], out_hbm.at[0],
                                  osem[b]).wait()

        def compute(b, g):
            p0 = lax.rem(g * CH, seq)
            U = 2

            def tok_body(i, carry):
                t0 = i * U
                tg = ttf_all[b, t0 >> 4, :]
                toks = []
                for u in range(U):
                    t = t0 + u
                    pt = p0 + t
                    p = pt - jnp.where(pt >= seq, seq, 0)
                    tts = _permute16(tg, jnp.broadcast_to(t & (L - 1), (L,)))
                    vs = []
                    acc = None
                    accsq = None
                    for j in range(J):
                        v = (rowbuf[b, t, pl.ds(L * j, L)]
                             + posadj_v[p, pl.ds(L * j, L)]
                             + tts * d8[j])
                        vs.append(v)
                        acc = v if acc is None else acc + v
                        accsq = v * v if accsq is None else accsq + v * v
                    toks.append((t, vs, acc, accsq))
                for t, vs, acc, accsq in toks:
                    mean_v = _hsum16(acc, perms) * (1.0 / H)
                    var_v = _hsum16(accsq, perms) * (1.0 / H) - mean_v * mean_v
                    inv_v = _rsqrt16(var_v + 1e-5)
                    for j in range(J):
                        rowbuf[b, t, pl.ds(L * j, L)] = ((vs[j] - mean_v)
                                                         * inv_v * g8[j]
                                                         + b8[j])
                return carry

            lax.fori_loop(0, CH // U, tok_body, 0)
            pltpu.async_copy(rowbuf.at[b], out_hbm.at[row0 + g], osem[b])

        def step(g, b, do_issue, do_owait):
            if do_issue:
                b3 = (b + 3) % NB
                if do_owait:
                    owait(b3)
                prep(g + 3, b3)
            gwait(b)
            compute(b, g)

        for gg in range(NB - 1):
            prep(gg, gg)
        step(0, 0, True, False)
        step(1, 1, True, True)
        step(2, 2, True, True)
        step(3, 3, True, True)

        def outer(go, carry):
            g0 = go * NB
            for b in range(NB):
                step(g0 + b, b, True, True)
            return carry

        ngroups = nchunk // NB
        lax.fori_loop(1, ngroups - 1, outer, 0)

        for g in range((ngroups - 1) * NB, nchunk):
            b = g % NB
            step(g, b, g + 3 < nchunk, True)
        for b in range(NB):
            owait(b)

    return sc_kernel(ids2, ttf3, tok_table, posadj, dseg, gamma, beta)


def kernel(input_ids, token_type_ids, tok_table, pos_table, seg_table, gamma, beta):
    B, S = input_ids.shape
    H = tok_table.shape[1]
    n = B * S
    nrows = n // CH
    ids2 = input_ids.astype(jnp.int32).reshape(nrows, CH)
    ttf3 = token_type_ids.astype(jnp.float32).reshape(nrows, CH // L, L)
    posadj = pos_table[:S] + seg_table[0][None, :]
    dseg = seg_table[1] - seg_table[0]
    out = _run(ids2, ttf3, tok_table, posadj, dseg, gamma, beta, n, H, S)
    return out.reshape(B, S, H)


# speedup vs baseline: 5.5043x; 1.1756x over previous
import functools

import jax
import jax.numpy as jnp
from jax import lax
from jax.experimental import pallas as pl
from jax.experimental.pallas import tpu as pltpu
from jax.experimental.pallas import tpu_sc as plsc

NC = 2
NS = 16
NW = NC * NS
L = 16
CH = 128
NB = 4

_GATHER_DNUMS = lax.GatherDimensionNumbers(
    offset_dims=(), collapsed_slice_dims=(0,), start_index_map=(0,))


def _permute16(v, p):
    return lax.gather(v, p[:, None], _GATHER_DNUMS, slice_sizes=(1,),
                      mode=lax.GatherScatterMode.PROMISE_IN_BOUNDS)


def _hsum16(v, perms):
    for p in perms:
        v = v + _permute16(v, p)
    return v


def _rsqrt16(x):
    i = lax.bitcast_convert_type(x, jnp.int32)
    y = lax.bitcast_convert_type(jnp.int32(0x5F3759DF) - (i >> 1), jnp.float32)
    for _ in range(3):
        y = y * (1.5 - 0.5 * x * y * y)
    return y


@functools.partial(jax.jit, static_argnums=(7, 8, 9))
def _run(ids2, ttf3, tok_table, posadj, dseg, gamma, beta,
         n_tokens, hidden, seq):
    H = hidden
    J = H // L
    tpw = n_tokens // NW
    nchunk = tpw // CH
    nrows = n_tokens // CH

    mesh = plsc.VectorSubcoreMesh(core_axis_name="c", subcore_axis_name="s")

    @functools.partial(
        pl.kernel,
        mesh=mesh,
        out_type=jax.ShapeDtypeStruct((nrows, CH, H), jnp.float32),
        scratch_types=[
            pltpu.VMEM((NB, CH), jnp.int32),
            pltpu.VMEM((NB, CH // L, L), jnp.float32),
            pltpu.VMEM((NB, CH, H), jnp.float32),
            pltpu.VMEM((seq, H), jnp.float32),
            pltpu.VMEM((H,), jnp.float32),
            pltpu.VMEM((H,), jnp.float32),
            pltpu.VMEM((H,), jnp.float32),
        ] + [pltpu.SemaphoreType.DMA] * (2 * NB),
    )
    def sc_kernel(ids_hbm, ttf_hbm, tok_hbm, posadj_hbm, dseg_hbm,
                  gamma_hbm, beta_hbm, out_hbm,
                  idx_all, ttf_all, rowbuf, posadj_v, dseg_v, gamma_v, beta_v,
                  *sems):
        gsem = sems[:NB]
        osem = sems[NB:]
        wid = lax.axis_index("s") * NC + lax.axis_index("c")
        row0 = wid * nchunk
        pltpu.sync_copy(posadj_hbm, posadj_v)
        pltpu.sync_copy(dseg_hbm, dseg_v)
        pltpu.sync_copy(gamma_hbm, gamma_v)
        pltpu.sync_copy(beta_hbm, beta_v)
        g8 = [gamma_v[pl.ds(L * j, L)] for j in range(J)]
        b8 = [beta_v[pl.ds(L * j, L)] for j in range(J)]
        d8 = [dseg_v[pl.ds(L * j, L)] for j in range(J)]
        lane = lax.iota(jnp.int32, L)
        perms = [lane ^ s for s in (8, 4, 2, 1)]

        def prep(g3, b3):
            r = row0 + g3
            pltpu.sync_copy(ids_hbm.at[r], idx_all.at[b3])
            pltpu.sync_copy(ttf_hbm.at[r], ttf_all.at[b3])
            pltpu.async_copy(tok_hbm.at[idx_all.at[b3]], rowbuf.at[b3],
                             gsem[b3])

        def gwait(b):
            pltpu.make_async_copy(tok_hbm.at[idx_all.at[b]], rowbuf.at[b],
                                  gsem[b]).wait()

        def owait(b):
            pltpu.make_async_copy(rowbuf.at[b], out_hbm.at[0],
                                  osem[b]).wait()

        def compute(b, g):
            p0 = lax.rem(g * CH, seq)
            U = 4

            def tok_body(i, carry):
                t0 = i * U
                tg = ttf_all[b, t0 >> 4, :]
                toks = []
                for u in range(U):
                    t = t0 + u
                    pt = p0 + t
                    p = pt - jnp.where(pt >= seq, seq, 0)
                    tts = _permute16(tg, jnp.broadcast_to(t & (L - 1), (L,)))
                    vs = []
                    acc = None
                    accsq = None
                    for j in range(J):
                        v = (rowbuf[b, t, pl.ds(L * j, L)]
                             + posadj_v[p, pl.ds(L * j, L)]
                             + tts * d8[j])
                        vs.append(v)
                        acc = v if acc is None else acc + v
                        accsq = v * v if accsq is None else accsq + v * v
                    toks.append((t, vs, acc, accsq))
                for t, vs, acc, accsq in toks:
                    mean_v = _hsum16(acc, perms) * (1.0 / H)
                    var_v = _hsum16(accsq, perms) * (1.0 / H) - mean_v * mean_v
                    inv_v = _rsqrt16(var_v + 1e-5)
                    for j in range(J):
                        rowbuf[b, t, pl.ds(L * j, L)] = ((vs[j] - mean_v)
                                                         * inv_v * g8[j]
                                                         + b8[j])
                return carry

            lax.fori_loop(0, CH // U, tok_body, 0)
            pltpu.async_copy(rowbuf.at[b], out_hbm.at[row0 + g], osem[b])

        def step(g, b, do_issue, do_owait):
            if do_issue:
                b3 = (b + 3) % NB
                if do_owait:
                    owait(b3)
                prep(g + 3, b3)
            gwait(b)
            compute(b, g)

        for gg in range(NB - 1):
            prep(gg, gg)
        step(0, 0, True, False)
        step(1, 1, True, True)
        step(2, 2, True, True)
        step(3, 3, True, True)

        def outer(go, carry):
            g0 = go * NB
            for b in range(NB):
                step(g0 + b, b, True, True)
            return carry

        ngroups = nchunk // NB
        lax.fori_loop(1, ngroups - 1, outer, 0)

        for g in range((ngroups - 1) * NB, nchunk):
            b = g % NB
            step(g, b, g + 3 < nchunk, True)
        for b in range(NB):
            owait(b)

    return sc_kernel(ids2, ttf3, tok_table, posadj, dseg, gamma, beta)


def kernel(input_ids, token_type_ids, tok_table, pos_table, seg_table, gamma, beta):
    B, S = input_ids.shape
    H = tok_table.shape[1]
    n = B * S
    nrows = n // CH
    ids2 = input_ids.astype(jnp.int32).reshape(nrows, CH)
    ttf3 = token_type_ids.astype(jnp.float32).reshape(nrows, CH // L, L)
    posadj = pos_table[:S] + seg_table[0][None, :]
    dseg = seg_table[1] - seg_table[0]
    out = _run(ids2, ttf3, tok_table, posadj, dseg, gamma, beta, n, H, S)
    return out.reshape(B, S, H)
